# baseline (speedup 1.0000x reference)
import math

import jax
import jax.numpy as jnp
import numpy as np
from jax.experimental import pallas as pl
from jax.experimental.pallas import tpu as pltpu

SIGMA_DATA = 1.0
EMBED_DIM = 256
HALF = EMBED_DIM // 2
_TWO_PI = 2.0 * math.pi


def _upsample_kernel(sig_ref, low_ref, colrep_ref, rowrep_ref, up_ref):
    b = pl.program_id(0)
    H = rowrep_ref.shape[1]
    NB = low_ref.shape[0]
    CH = low_ref.shape[1]
    C = CH // H

    for k in range(NB):
        sig = sig_ref[b * NB + k]
        c_in = jax.lax.rsqrt(sig * sig + SIGMA_DATA * SIGMA_DATA)
        x = (low_ref[k].astype(jnp.float32) * c_in
             ).astype(jnp.bfloat16)
        y = jnp.dot(x, colrep_ref[...],
                    preferred_element_type=jnp.float32)
        yb = y.astype(jnp.bfloat16)
        for c in range(C):
            up_ref[k, 2 * H * c:2 * H * (c + 1), :] = jnp.dot(
                rowrep_ref[...], yb[H * c:H * (c + 1), :],
                preferred_element_type=jnp.float32).astype(up_ref.dtype)


def _mapping_kernel(sig_ref, w_ref, pooler_ref, map_ref):
    B = map_ref.shape[0]
    idx = jax.lax.broadcasted_iota(jnp.int32, (B, 1), 0)
    sv = jnp.zeros((B, 1), jnp.float32)
    for i in range(B):
        sv = jnp.where(idx == i, sig_ref[i], sv)
    u = 1.0 + sv
    log1p_sig = jnp.log(u) - ((u - 1.0) - sv) / u
    f = (_TWO_PI * log1p_sig) * w_ref[...]
    map_ref[:, :HALF] = jnp.cos(f)
    map_ref[:, HALF:EMBED_DIM] = jnp.sin(f)
    map_ref[:, EMBED_DIM:] = pooler_ref[...].astype(jnp.float32)


def kernel(inputs, sigma, low_res, low_res_sigma, cross_cond,
           cross_cond_padding, pooler, fourier_weight):
    B, C, H, W = low_res.shape
    P = pooler.shape[1]
    out_dtype = low_res.dtype

    m = np.arange(2 * W)[None, :]
    colrep = jnp.asarray((np.arange(W)[:, None] == m // 2), dtype=jnp.bfloat16)
    r = np.arange(2 * H)[:, None]
    rowrep = jnp.asarray((r // 2 == np.arange(H)[None, :]), dtype=jnp.bfloat16)

    low2d = low_res.reshape(B, C * H, W)

    sig32 = low_res_sigma.astype(jnp.float32)

    up3d = pl.pallas_call(
        _upsample_kernel,
        out_shape=jax.ShapeDtypeStruct((B, C * 2 * H, 2 * W), out_dtype),
        grid=(B // 16, 1),
        in_specs=[
            pl.BlockSpec(memory_space=pltpu.MemorySpace.SMEM),
            pl.BlockSpec((16, C * H, W), lambda b, r: (b, 0, 0)),
            pl.BlockSpec((W, 2 * W), lambda b, r: (0, 0)),
            pl.BlockSpec((2 * H, H), lambda b, r: (0, 0)),
        ],
        out_specs=pl.BlockSpec((16, C * 2 * H, 2 * W),
                               lambda b, r: (b, 0, 0)),
        compiler_params=pltpu.CompilerParams(
            dimension_semantics=("parallel", "parallel"),
            vmem_limit_bytes=60 << 20),
    )(sig32, low2d, colrep, rowrep)
    up = up3d.reshape(B, C, 2 * H, 2 * W)

    mapping_cond = pl.pallas_call(
        _mapping_kernel,
        out_shape=jax.ShapeDtypeStruct((B, EMBED_DIM + P), jnp.float32),
        in_specs=[
            pl.BlockSpec(memory_space=pltpu.MemorySpace.SMEM),
            pl.BlockSpec((1, HALF), lambda: (0, 0)),
            pl.BlockSpec((B, P), lambda: (0, 0)),
        ],
        out_specs=pl.BlockSpec((B, EMBED_DIM + P), lambda: (0, 0)),
    )(sig32, fourier_weight.astype(jnp.float32).reshape(1, HALF),
      pooler.astype(jnp.float32))

    return {
        "inputs": inputs,
        "sigma": sigma,
        "unet_cond": up,
        "mapping_cond": mapping_cond,
        "cross_cond": cross_cond,
        "cross_cond_padding": cross_cond_padding,
    }


# speedup vs baseline: 1.2542x; 1.2542x over previous
import math

import jax
import jax.numpy as jnp
import numpy as np
from jax.experimental import pallas as pl
from jax.experimental.pallas import tpu as pltpu

SIGMA_DATA = 1.0
EMBED_DIM = 256
HALF = EMBED_DIM // 2
_TWO_PI = 2.0 * math.pi


def _upsample_kernel(sig_ref, low_ref, colrep_ref, rowrep_ref, up_ref):
    b = pl.program_id(0)
    H = rowrep_ref.shape[1]
    NB = low_ref.shape[0]
    CH = low_ref.shape[1]
    C = CH // H

    for k in range(NB):
        sig = sig_ref[b * NB + k]
        c_in = jax.lax.rsqrt(sig * sig + SIGMA_DATA * SIGMA_DATA)
        x = (low_ref[k].astype(jnp.float32) * c_in
             ).astype(jnp.bfloat16)
        y = jnp.dot(x, colrep_ref[...],
                    preferred_element_type=jnp.float32)
        yb = y.astype(jnp.bfloat16)
        for c in range(C):
            up_ref[k, 2 * H * c:2 * H * (c + 1), :] = jnp.dot(
                rowrep_ref[...], yb[H * c:H * (c + 1), :],
                preferred_element_type=jnp.float32).astype(up_ref.dtype)


def _mapping_kernel(sig_ref, w_ref, pooler_ref, map_ref):
    B = map_ref.shape[0]
    idx = jax.lax.broadcasted_iota(jnp.int32, (B, 1), 0)
    sv = jnp.zeros((B, 1), jnp.float32)
    for i in range(B):
        sv = jnp.where(idx == i, sig_ref[i], sv)
    u = 1.0 + sv
    log1p_sig = jnp.log(u) - ((u - 1.0) - sv) / u
    f = (_TWO_PI * log1p_sig) * w_ref[...]
    map_ref[:, :HALF] = jnp.cos(f)
    map_ref[:, HALF:EMBED_DIM] = jnp.sin(f)
    map_ref[:, EMBED_DIM:] = pooler_ref[...].astype(jnp.float32)


def kernel(inputs, sigma, low_res, low_res_sigma, cross_cond,
           cross_cond_padding, pooler, fourier_weight):
    B, C, H, W = low_res.shape
    P = pooler.shape[1]
    out_dtype = low_res.dtype

    m = np.arange(2 * W)[None, :]
    colrep = jnp.asarray((np.arange(W)[:, None] == m // 2), dtype=jnp.bfloat16)
    r = np.arange(2 * H)[:, None]
    rowrep = jnp.asarray((r // 2 == np.arange(H)[None, :]), dtype=jnp.bfloat16)

    low2d = low_res.reshape(B, C * H, W)

    sig32 = low_res_sigma.astype(jnp.float32)

    up3d = pl.pallas_call(
        _upsample_kernel,
        out_shape=jax.ShapeDtypeStruct((B, C * 2 * H, 2 * W), out_dtype),
        grid=(B // 8, 1),
        in_specs=[
            pl.BlockSpec(memory_space=pltpu.MemorySpace.SMEM),
            pl.BlockSpec((8, C * H, W), lambda b, r: (b, 0, 0)),
            pl.BlockSpec((W, 2 * W), lambda b, r: (0, 0)),
            pl.BlockSpec((2 * H, H), lambda b, r: (0, 0)),
        ],
        out_specs=pl.BlockSpec((8, C * 2 * H, 2 * W),
                               lambda b, r: (b, 0, 0)),
        compiler_params=pltpu.CompilerParams(
            dimension_semantics=("parallel", "parallel"),
            vmem_limit_bytes=32 << 20),
    )(sig32, low2d, colrep, rowrep)
    up = up3d.reshape(B, C, 2 * H, 2 * W)

    mapping_cond = pl.pallas_call(
        _mapping_kernel,
        out_shape=jax.ShapeDtypeStruct((B, EMBED_DIM + P), jnp.float32),
        in_specs=[
            pl.BlockSpec(memory_space=pltpu.MemorySpace.SMEM),
            pl.BlockSpec((1, HALF), lambda: (0, 0)),
            pl.BlockSpec((B, P), lambda: (0, 0)),
        ],
        out_specs=pl.BlockSpec((B, EMBED_DIM + P), lambda: (0, 0)),
    )(sig32, fourier_weight.astype(jnp.float32).reshape(1, HALF),
      pooler.astype(jnp.float32))

    return {
        "inputs": inputs,
        "sigma": sigma,
        "unet_cond": up,
        "mapping_cond": mapping_cond,
        "cross_cond": cross_cond,
        "cross_cond_padding": cross_cond_padding,
    }
